# baseline (speedup 1.0000x reference)
import functools

import jax
import jax.numpy as jnp
from jax import lax
from jax.experimental import pallas as pl
from jax.experimental.pallas import tpu as pltpu
from jax.experimental.pallas import tpu_sc as plsc

N = 10000
K = 16
C = 128
EPS = 1e-5
NK = float(N * K)

QBLK = 400
NQPAD = 10240
NBINS = 10240
NW = 32
RPW = NQPAD * K // 128 // NW
QPW = NQPAD // NW


PW = 10112
GN = PW // 128


def _knn_body(q_ref, pt_ref, out_ref):
    q = q_ref[...]
    pt = pt_ref[...]
    qsq = jnp.sum(q * q, axis=1, keepdims=True)
    psq = jnp.sum(pt * pt, axis=0, keepdims=True)
    d = qsq + psq - 2.0 * jnp.dot(q, pt, preferred_element_type=jnp.float32)
    iota = lax.broadcasted_iota(jnp.int32, d.shape, 1)
    gidx = iota >> 7
    g_io = lax.broadcasted_iota(jnp.int32, (QBLK, GN), 1)
    inf = jnp.float32(jnp.inf)
    gmin = jnp.concatenate(
        [jnp.min(d[:, g * 128:(g + 1) * 128], axis=1, keepdims=True)
         for g in range(GN)], axis=1)
    cols = []
    for _ in range(K):
        m = jnp.min(gmin, axis=1, keepdims=True)
        cstar = jnp.min(jnp.where(gmin == m, g_io, GN), axis=1)
        gmask = gidx == cstar[:, None]
        ii = jnp.min(jnp.where(gmask & (d == m), iota, jnp.int32(PW)), axis=1)
        cols.append(ii)
        d = jnp.where(iota == ii[:, None], inf, d)
        newgm = jnp.min(jnp.where(gmask, d, inf), axis=1)
        gmin = jnp.where(g_io == cstar[:, None], newgm[:, None], gmin)
    out_ref[...] = jnp.stack(cols, axis=1)


def _knn(coord):
    ptp = jnp.full((3, PW), 1.0e4, jnp.float32).at[:, :N].set(coord.T)
    return pl.pallas_call(
        _knn_body,
        grid=(N // QBLK,),
        in_specs=[
            pl.BlockSpec((QBLK, 3), lambda i: (i, 0)),
            pl.BlockSpec((3, PW), lambda i: (0, 0)),
        ],
        out_specs=pl.BlockSpec((QBLK, K), lambda i: (i, 0)),
        out_shape=jax.ShapeDtypeStruct((N, K), jnp.int32),
    )(coord, ptp)


def _counts_body(idx_hbm, out_hbm, idx_v, counts_v, tbuf_v, acc_v, shared):
    cid = lax.axis_index("c")
    sid = lax.axis_index("s")
    wid = cid * 16 + sid
    sl = NBINS // 16

    pltpu.sync_copy(idx_hbm.at[pl.ds(wid * RPW, RPW)], idx_v)

    zero16 = jnp.zeros((16,), jnp.int32)

    def zero_body(i, _):
        counts_v[pl.ds(i * 16, 16)] = zero16
        return 0

    lax.fori_loop(0, NBINS // 16, zero_body, 0)

    ones = jnp.ones((16,), jnp.int32)

    def hist_body(j, _):
        for g in range(8):
            plsc.addupdate_scatter(counts_v, [idx_v[j, g]], ones)
        return 0

    lax.fori_loop(0, RPW, hist_body, 0)

    pltpu.sync_copy(counts_v, shared.at[sid])
    plsc.subcore_barrier()

    def azero(i, _):
        acc_v[pl.ds(i * 16, 16)] = zero16
        return 0

    lax.fori_loop(0, sl // 16, azero, 0)
    for t in range(16):
        pltpu.sync_copy(shared.at[t, pl.ds(sid * sl, sl)], tbuf_v)

        def add_body(i, _):
            acc_v[pl.ds(i * 16, 16)] = (
                acc_v[pl.ds(i * 16, 16)] + tbuf_v[pl.ds(i * 16, 16)]
            )
            return 0

        lax.fori_loop(0, sl // 16, add_body, 0)

    pltpu.sync_copy(acc_v, out_hbm.at[cid, pl.ds(sid * sl, sl)])


@functools.lru_cache(maxsize=None)
def _counts_sc():
    return pl.kernel(
        _counts_body,
        out_type=jax.ShapeDtypeStruct((2, NBINS), jnp.int32),
        mesh=plsc.VectorSubcoreMesh(core_axis_name="c", subcore_axis_name="s"),
        compiler_params=pltpu.CompilerParams(needs_layout_passes=False),
        scratch_types=[
            pltpu.VMEM((RPW, 8, K), jnp.int32),
            pltpu.VMEM((NBINS,), jnp.int32),
            pltpu.VMEM((NBINS // 16,), jnp.int32),
            pltpu.VMEM((NBINS // 16,), jnp.int32),
            pltpu.VMEM_SHARED((16, NBINS), jnp.int32),
        ],
    )


def _mlp_body(x_ref, c_ref, w1_ref, b1_ref, g1_ref, e1_ref,
              w2_ref, b2_ref, g2_ref, e2_ref, out_ref):
    c = c_ref[...]

    def layer(x, w, b, g, e):
        z = jnp.dot(x, w, preferred_element_type=jnp.float32) + b
        s1 = jnp.dot(c, z, preferred_element_type=jnp.float32) * (1.0 / NK)
        s2 = jnp.dot(c, z * z, preferred_element_type=jnp.float32) * (1.0 / NK)
        var = s2 - s1 * s1
        return jnp.maximum((z - s1) * lax.rsqrt(var + EPS) * g + e, 0.0)

    h1 = layer(x_ref[...], w1_ref[...], b1_ref[...], g1_ref[...], e1_ref[...])
    out_ref[...] = layer(h1, w2_ref[...], b2_ref[...], g2_ref[...], e2_ref[...])


def _mlp(feat, c, w1, b1, g1, e1, w2, b2, g2, e2):
    full = lambda s: pl.BlockSpec(s, lambda: (0,) * len(s))
    row = lambda: full((1, C))
    return pl.pallas_call(
        _mlp_body,
        in_specs=[full((N, C)), full((1, N)), full((C, C)), row(), row(), row(),
                  full((C, C)), row(), row(), row()],
        out_specs=full((N, C)),
        out_shape=jax.ShapeDtypeStruct((N, C), jnp.float32),
    )(feat, c, w1, b1.reshape(1, C), g1.reshape(1, C), e1.reshape(1, C),
      w2, b2.reshape(1, C), g2.reshape(1, C), e2.reshape(1, C))


def _gather_max_body(h_hbm, idx_hbm, out_hbm, idx_v, rows_v, outb_v, sem):
    cid = lax.axis_index("c")
    sid = lax.axis_index("s")
    wid = cid * 16 + sid

    pltpu.sync_copy(idx_hbm.at[pl.ds(wid * RPW, RPW)], idx_v)

    def batch(b, _):
        pltpu.async_copy(h_hbm.at[idx_v.at[b]], rows_v, sem).wait()
        for q in range(8):
            for cc in range(C // 16):
                acc = rows_v[16 * q, pl.ds(cc * 16, 16)]
                for r in range(1, K):
                    acc = jnp.maximum(acc, rows_v[16 * q + r, pl.ds(cc * 16, 16)])
                outb_v[q, pl.ds(cc * 16, 16)] = acc
        pltpu.sync_copy(outb_v, out_hbm.at[pl.ds(wid * QPW + b * 8, 8)])
        return 0

    lax.fori_loop(0, RPW, batch, 0)


@functools.lru_cache(maxsize=None)
def _gather_max_sc():
    return pl.kernel(
        _gather_max_body,
        out_type=jax.ShapeDtypeStruct((NQPAD, C), jnp.float32),
        mesh=plsc.VectorSubcoreMesh(core_axis_name="c", subcore_axis_name="s"),
        compiler_params=pltpu.CompilerParams(needs_layout_passes=False),
        scratch_types=[
            pltpu.VMEM((RPW, 128), jnp.int32),
            pltpu.VMEM((128, C), jnp.float32),
            pltpu.VMEM((8, C), jnp.float32),
            pltpu.SemaphoreType.DMA,
        ],
    )


def kernel(coord, feat, W1, b1, g1, be1, W2, b2, g2, be2, offset):
    idx = _knn(coord)
    pad_row = jnp.arange(N, N + K, dtype=jnp.int32)
    idx_pad = jnp.concatenate(
        [idx, jnp.broadcast_to(pad_row, (NQPAD - N, K))], axis=0)
    cnt = _counts_sc()(idx_pad.reshape(NQPAD * K // 128, 8, K))
    c = (cnt[0, :N] + cnt[1, :N]).astype(jnp.float32).reshape(1, N)
    h2 = _mlp(feat, c, W1, b1, g1, be1, W2, b2, g2, be2)
    h2p = jnp.concatenate([h2, jnp.zeros((K, C), jnp.float32)], axis=0)
    out = _gather_max_sc()(h2p, idx_pad.reshape(NQPAD * K // 128, 128))
    return out[:N]


# speedup vs baseline: 1.4310x; 1.4310x over previous
import functools

import jax
import jax.numpy as jnp
from jax import lax
from jax.experimental import pallas as pl
from jax.experimental.pallas import tpu as pltpu
from jax.experimental.pallas import tpu_sc as plsc

N = 10000
K = 16
C = 128
EPS = 1e-5
NK = float(N * K)

QBLK = 400
NQPAD = 10240
NBINS = 10240
NW = 32
RPW = NQPAD * K // 128 // NW
QPW = NQPAD // NW


PW = 10112
GN = PW // 128


def _knn_body(q_ref, pt_ref, out_ref):
    q = q_ref[...]
    pt = pt_ref[...]
    qsq = jnp.sum(q * q, axis=1, keepdims=True)
    psq = jnp.sum(pt * pt, axis=0, keepdims=True)
    d = qsq + psq - 2.0 * jnp.dot(q, pt, preferred_element_type=jnp.float32)
    iota = lax.broadcasted_iota(jnp.int32, d.shape, 1)
    inf = jnp.float32(jnp.inf)
    cols = []
    for _ in range(K):
        m = jnp.min(d, axis=1, keepdims=True)
        ii = jnp.min(jnp.where(d == m, iota, jnp.int32(PW)), axis=1)
        cols.append(ii)
        d = jnp.where(iota == ii[:, None], inf, d)
    out_ref[...] = jnp.stack(cols, axis=1)


def _knn(coord):
    ptp = jnp.full((3, PW), 1.0e4, jnp.float32).at[:, :N].set(coord.T)
    return pl.pallas_call(
        _knn_body,
        grid=(N // QBLK,),
        in_specs=[
            pl.BlockSpec((QBLK, 3), lambda i: (i, 0)),
            pl.BlockSpec((3, PW), lambda i: (0, 0)),
        ],
        out_specs=pl.BlockSpec((QBLK, K), lambda i: (i, 0)),
        out_shape=jax.ShapeDtypeStruct((N, K), jnp.int32),
    )(coord, ptp)


def _counts_body(idx_hbm, out_hbm, idx_v, counts_v, tbuf_v, acc_v, shared):
    cid = lax.axis_index("c")
    sid = lax.axis_index("s")
    wid = cid * 16 + sid
    sl = NBINS // 16

    pltpu.sync_copy(idx_hbm.at[pl.ds(wid * RPW, RPW)], idx_v)

    zero16 = jnp.zeros((16,), jnp.int32)

    def zero_body(i, _):
        counts_v[pl.ds(i * 16, 16)] = zero16
        return 0

    lax.fori_loop(0, NBINS // 16, zero_body, 0)

    ones = jnp.ones((16,), jnp.int32)

    def hist_body(j, _):
        for g in range(8):
            plsc.addupdate_scatter(counts_v, [idx_v[j, g]], ones)
        return 0

    lax.fori_loop(0, RPW, hist_body, 0)

    pltpu.sync_copy(counts_v, shared.at[sid])
    plsc.subcore_barrier()

    def azero(i, _):
        acc_v[pl.ds(i * 16, 16)] = zero16
        return 0

    lax.fori_loop(0, sl // 16, azero, 0)
    for t in range(16):
        pltpu.sync_copy(shared.at[t, pl.ds(sid * sl, sl)], tbuf_v)

        def add_body(i, _):
            acc_v[pl.ds(i * 16, 16)] = (
                acc_v[pl.ds(i * 16, 16)] + tbuf_v[pl.ds(i * 16, 16)]
            )
            return 0

        lax.fori_loop(0, sl // 16, add_body, 0)

    pltpu.sync_copy(acc_v, out_hbm.at[cid, pl.ds(sid * sl, sl)])


@functools.lru_cache(maxsize=None)
def _counts_sc():
    return pl.kernel(
        _counts_body,
        out_type=jax.ShapeDtypeStruct((2, NBINS), jnp.int32),
        mesh=plsc.VectorSubcoreMesh(core_axis_name="c", subcore_axis_name="s"),
        compiler_params=pltpu.CompilerParams(needs_layout_passes=False),
        scratch_types=[
            pltpu.VMEM((RPW, 8, K), jnp.int32),
            pltpu.VMEM((NBINS,), jnp.int32),
            pltpu.VMEM((NBINS // 16,), jnp.int32),
            pltpu.VMEM((NBINS // 16,), jnp.int32),
            pltpu.VMEM_SHARED((16, NBINS), jnp.int32),
        ],
    )


def _mlp_body(x_ref, c_ref, w1_ref, b1_ref, g1_ref, e1_ref,
              w2_ref, b2_ref, g2_ref, e2_ref, out_ref):
    c = c_ref[...]

    def layer(x, w, b, g, e):
        z = jnp.dot(x, w, preferred_element_type=jnp.float32) + b
        s1 = jnp.dot(c, z, preferred_element_type=jnp.float32) * (1.0 / NK)
        s2 = jnp.dot(c, z * z, preferred_element_type=jnp.float32) * (1.0 / NK)
        var = s2 - s1 * s1
        return jnp.maximum((z - s1) * lax.rsqrt(var + EPS) * g + e, 0.0)

    h1 = layer(x_ref[...], w1_ref[...], b1_ref[...], g1_ref[...], e1_ref[...])
    out_ref[...] = layer(h1, w2_ref[...], b2_ref[...], g2_ref[...], e2_ref[...])


def _mlp(feat, c, w1, b1, g1, e1, w2, b2, g2, e2):
    full = lambda s: pl.BlockSpec(s, lambda: (0,) * len(s))
    row = lambda: full((1, C))
    return pl.pallas_call(
        _mlp_body,
        in_specs=[full((N, C)), full((1, N)), full((C, C)), row(), row(), row(),
                  full((C, C)), row(), row(), row()],
        out_specs=full((N, C)),
        out_shape=jax.ShapeDtypeStruct((N, C), jnp.float32),
    )(feat, c, w1, b1.reshape(1, C), g1.reshape(1, C), e1.reshape(1, C),
      w2, b2.reshape(1, C), g2.reshape(1, C), e2.reshape(1, C))


def _gather_max_body(h_hbm, idx_hbm, out_hbm, idx_v, rows0, rows1, outb_v,
                     sem0, sem1):
    cid = lax.axis_index("c")
    sid = lax.axis_index("s")
    wid = cid * 16 + sid

    pltpu.sync_copy(idx_hbm.at[pl.ds(wid * RPW, RPW)], idx_v)

    def reduce_store(rows_v, b):
        for q in range(8):
            for cc in range(C // 16):
                acc = rows_v[16 * q, pl.ds(cc * 16, 16)]
                for r in range(1, K):
                    acc = jnp.maximum(acc, rows_v[16 * q + r, pl.ds(cc * 16, 16)])
                outb_v[q, pl.ds(cc * 16, 16)] = acc
        pltpu.sync_copy(outb_v, out_hbm.at[pl.ds(wid * QPW + b * 8, 8)])

    pltpu.async_copy(h_hbm.at[idx_v.at[0]], rows0, sem0)

    def pair(i, _):
        b0 = 2 * i
        pltpu.async_copy(h_hbm.at[idx_v.at[b0 + 1]], rows1, sem1)
        pltpu.make_async_copy(h_hbm.at[idx_v.at[b0]], rows0, sem0).wait()
        reduce_store(rows0, b0)

        @pl.when(i < RPW // 2 - 1)
        def _():
            pltpu.async_copy(h_hbm.at[idx_v.at[b0 + 2]], rows0, sem0)

        pltpu.make_async_copy(h_hbm.at[idx_v.at[b0 + 1]], rows1, sem1).wait()
        reduce_store(rows1, b0 + 1)
        return 0

    lax.fori_loop(0, RPW // 2, pair, 0)


@functools.lru_cache(maxsize=None)
def _gather_max_sc():
    return pl.kernel(
        _gather_max_body,
        out_type=jax.ShapeDtypeStruct((NQPAD, C), jnp.float32),
        mesh=plsc.VectorSubcoreMesh(core_axis_name="c", subcore_axis_name="s"),
        compiler_params=pltpu.CompilerParams(needs_layout_passes=False),
        scratch_types=[
            pltpu.VMEM((RPW, 128), jnp.int32),
            pltpu.VMEM((128, C), jnp.float32),
            pltpu.VMEM((128, C), jnp.float32),
            pltpu.VMEM((8, C), jnp.float32),
            pltpu.SemaphoreType.DMA,
            pltpu.SemaphoreType.DMA,
        ],
    )


def kernel(coord, feat, W1, b1, g1, be1, W2, b2, g2, be2, offset):
    idx = _knn(coord)
    pad_row = jnp.arange(N, N + K, dtype=jnp.int32)
    idx_pad = jnp.concatenate(
        [idx, jnp.broadcast_to(pad_row, (NQPAD - N, K))], axis=0)
    cnt = _counts_sc()(idx_pad.reshape(NQPAD * K // 128, 8, K))
    c = (cnt[0, :N] + cnt[1, :N]).astype(jnp.float32).reshape(1, N)
    h2 = _mlp(feat, c, W1, b1, g1, be1, W2, b2, g2, be2)
    h2p = jnp.concatenate([h2, jnp.zeros((K, C), jnp.float32)], axis=0)
    out = _gather_max_sc()(h2p, idx_pad.reshape(NQPAD * K // 128, 128))
    return out[:N]


# speedup vs baseline: 1.5328x; 1.0711x over previous
import functools

import jax
import jax.numpy as jnp
from jax import lax
from jax.experimental import pallas as pl
from jax.experimental.pallas import tpu as pltpu
from jax.experimental.pallas import tpu_sc as plsc

N = 10000
K = 16
C = 128
EPS = 1e-5
NK = float(N * K)

QBLK = 1000
NQPAD = 10240
NBINS = 10240
NW = 32
RPW = NQPAD * K // 128 // NW
QPW = NQPAD // NW


PW = 10112
GN = PW // 128


def _knn_body(q_ref, pt_ref, out_ref):
    q = q_ref[...]
    pt = pt_ref[...]
    qsq = jnp.sum(q * q, axis=1, keepdims=True)
    psq = jnp.sum(pt * pt, axis=0, keepdims=True)
    d = qsq + psq - 2.0 * jnp.dot(q, pt, preferred_element_type=jnp.float32)
    iota = lax.broadcasted_iota(jnp.int32, d.shape, 1)
    inf = jnp.float32(jnp.inf)
    cols = []
    for _ in range(K):
        m = jnp.min(d, axis=1, keepdims=True)
        ii = jnp.min(jnp.where(d == m, iota, jnp.int32(PW)), axis=1)
        cols.append(ii)
        d = jnp.where(iota == ii[:, None], inf, d)
    out_ref[...] = jnp.stack(cols, axis=1)


def _knn(coord):
    ptp = jnp.full((3, PW), 1.0e4, jnp.float32).at[:, :N].set(coord.T)
    return pl.pallas_call(
        _knn_body,
        grid=(N // QBLK,),
        in_specs=[
            pl.BlockSpec((QBLK, 3), lambda i: (i, 0)),
            pl.BlockSpec((3, PW), lambda i: (0, 0)),
        ],
        out_specs=pl.BlockSpec((QBLK, K), lambda i: (i, 0)),
        out_shape=jax.ShapeDtypeStruct((N, K), jnp.int32),
    )(coord, ptp)


def _counts_body(idx_hbm, out_hbm, idx_v, counts_v, tbuf_v, acc_v, shared):
    cid = lax.axis_index("c")
    sid = lax.axis_index("s")
    wid = cid * 16 + sid
    sl = NBINS // 16

    pltpu.sync_copy(idx_hbm.at[pl.ds(wid * RPW, RPW)], idx_v)

    zero16 = jnp.zeros((16,), jnp.int32)

    def zero_body(i, _):
        counts_v[pl.ds(i * 16, 16)] = zero16
        return 0

    lax.fori_loop(0, NBINS // 16, zero_body, 0)

    ones = jnp.ones((16,), jnp.int32)

    def hist_body(j, _):
        for g in range(8):
            plsc.addupdate_scatter(counts_v, [idx_v[j, g]], ones)
        return 0

    lax.fori_loop(0, RPW, hist_body, 0)

    pltpu.sync_copy(counts_v, shared.at[sid])
    plsc.subcore_barrier()

    def azero(i, _):
        acc_v[pl.ds(i * 16, 16)] = zero16
        return 0

    lax.fori_loop(0, sl // 16, azero, 0)
    for t in range(16):
        pltpu.sync_copy(shared.at[t, pl.ds(sid * sl, sl)], tbuf_v)

        def add_body(i, _):
            acc_v[pl.ds(i * 16, 16)] = (
                acc_v[pl.ds(i * 16, 16)] + tbuf_v[pl.ds(i * 16, 16)]
            )
            return 0

        lax.fori_loop(0, sl // 16, add_body, 0)

    pltpu.sync_copy(acc_v, out_hbm.at[cid, pl.ds(sid * sl, sl)])


@functools.lru_cache(maxsize=None)
def _counts_sc():
    return pl.kernel(
        _counts_body,
        out_type=jax.ShapeDtypeStruct((2, NBINS), jnp.int32),
        mesh=plsc.VectorSubcoreMesh(core_axis_name="c", subcore_axis_name="s"),
        compiler_params=pltpu.CompilerParams(needs_layout_passes=False),
        scratch_types=[
            pltpu.VMEM((RPW, 8, K), jnp.int32),
            pltpu.VMEM((NBINS,), jnp.int32),
            pltpu.VMEM((NBINS // 16,), jnp.int32),
            pltpu.VMEM((NBINS // 16,), jnp.int32),
            pltpu.VMEM_SHARED((16, NBINS), jnp.int32),
        ],
    )


def _mlp_body(x_ref, c_ref, w1_ref, b1_ref, g1_ref, e1_ref,
              w2_ref, b2_ref, g2_ref, e2_ref, out_ref):
    c = c_ref[...]

    def layer(x, w, b, g, e):
        z = jnp.dot(x, w, preferred_element_type=jnp.float32) + b
        s1 = jnp.dot(c, z, preferred_element_type=jnp.float32) * (1.0 / NK)
        s2 = jnp.dot(c, z * z, preferred_element_type=jnp.float32) * (1.0 / NK)
        var = s2 - s1 * s1
        return jnp.maximum((z - s1) * lax.rsqrt(var + EPS) * g + e, 0.0)

    h1 = layer(x_ref[...], w1_ref[...], b1_ref[...], g1_ref[...], e1_ref[...])
    out_ref[...] = layer(h1, w2_ref[...], b2_ref[...], g2_ref[...], e2_ref[...])


def _mlp(feat, c, w1, b1, g1, e1, w2, b2, g2, e2):
    full = lambda s: pl.BlockSpec(s, lambda: (0,) * len(s))
    row = lambda: full((1, C))
    return pl.pallas_call(
        _mlp_body,
        in_specs=[full((N, C)), full((1, N)), full((C, C)), row(), row(), row(),
                  full((C, C)), row(), row(), row()],
        out_specs=full((N, C)),
        out_shape=jax.ShapeDtypeStruct((N, C), jnp.float32),
    )(feat, c, w1, b1.reshape(1, C), g1.reshape(1, C), e1.reshape(1, C),
      w2, b2.reshape(1, C), g2.reshape(1, C), e2.reshape(1, C))


def _gather_max_body(h_hbm, idx_hbm, out_hbm, idx_v, rows0, rows1, outb_v,
                     sem0, sem1):
    cid = lax.axis_index("c")
    sid = lax.axis_index("s")
    wid = cid * 16 + sid

    pltpu.sync_copy(idx_hbm.at[pl.ds(wid * RPW, RPW)], idx_v)

    def reduce_store(rows_v, b):
        for q in range(8):
            for cc in range(C // 16):
                acc = rows_v[16 * q, pl.ds(cc * 16, 16)]
                for r in range(1, K):
                    acc = jnp.maximum(acc, rows_v[16 * q + r, pl.ds(cc * 16, 16)])
                outb_v[q, pl.ds(cc * 16, 16)] = acc
        pltpu.sync_copy(outb_v, out_hbm.at[pl.ds(wid * QPW + b * 8, 8)])

    pltpu.async_copy(h_hbm.at[idx_v.at[0]], rows0, sem0)

    def pair(i, _):
        b0 = 2 * i
        pltpu.async_copy(h_hbm.at[idx_v.at[b0 + 1]], rows1, sem1)
        pltpu.make_async_copy(h_hbm.at[idx_v.at[b0]], rows0, sem0).wait()
        reduce_store(rows0, b0)

        @pl.when(i < RPW // 2 - 1)
        def _():
            pltpu.async_copy(h_hbm.at[idx_v.at[b0 + 2]], rows0, sem0)

        pltpu.make_async_copy(h_hbm.at[idx_v.at[b0 + 1]], rows1, sem1).wait()
        reduce_store(rows1, b0 + 1)
        return 0

    lax.fori_loop(0, RPW // 2, pair, 0)


@functools.lru_cache(maxsize=None)
def _gather_max_sc():
    return pl.kernel(
        _gather_max_body,
        out_type=jax.ShapeDtypeStruct((NQPAD, C), jnp.float32),
        mesh=plsc.VectorSubcoreMesh(core_axis_name="c", subcore_axis_name="s"),
        compiler_params=pltpu.CompilerParams(needs_layout_passes=False),
        scratch_types=[
            pltpu.VMEM((RPW, 128), jnp.int32),
            pltpu.VMEM((128, C), jnp.float32),
            pltpu.VMEM((128, C), jnp.float32),
            pltpu.VMEM((8, C), jnp.float32),
            pltpu.SemaphoreType.DMA,
            pltpu.SemaphoreType.DMA,
        ],
    )


def kernel(coord, feat, W1, b1, g1, be1, W2, b2, g2, be2, offset):
    idx = _knn(coord)
    pad_row = jnp.arange(N, N + K, dtype=jnp.int32)
    idx_pad = jnp.concatenate(
        [idx, jnp.broadcast_to(pad_row, (NQPAD - N, K))], axis=0)
    cnt = _counts_sc()(idx_pad.reshape(NQPAD * K // 128, 8, K))
    c = (cnt[0, :N] + cnt[1, :N]).astype(jnp.float32).reshape(1, N)
    h2 = _mlp(feat, c, W1, b1, g1, be1, W2, b2, g2, be2)
    h2p = jnp.concatenate([h2, jnp.zeros((K, C), jnp.float32)], axis=0)
    out = _gather_max_sc()(h2p, idx_pad.reshape(NQPAD * K // 128, 128))
    return out[:N]
